# Initial kernel scaffold; baseline (speedup 1.0000x reference)
import jax
import jax.numpy as jnp
from jax.experimental import pallas as pl


def kernel(input_ids, weight):
    raise NotImplementedError("write your pallas kernel here")


# speedup vs baseline: 1.6426x; 1.6426x over previous
import functools

import jax
import jax.numpy as jnp
from jax import lax
from jax.experimental import pallas as pl
from jax.experimental.pallas import tpu as pltpu
from jax.experimental.pallas import tpu_sc as plsc

_BATCH = 4
_SEQ = 4096
_HIDDEN = 1024
_N = _BATCH * _SEQ
_NC = 2
_NS = 16
_NW = _NC * _NS
_PER_W = _N // _NW
_CHUNK = 32
_NCH = _PER_W // _CHUNK

_mesh = plsc.VectorSubcoreMesh(
    core_axis_name="c", subcore_axis_name="s", num_cores=_NC, num_subcores=_NS
)


@functools.partial(
    pl.kernel,
    out_type=jax.ShapeDtypeStruct((_N, _HIDDEN), jnp.float32),
    mesh=_mesh,
    scratch_types=[
        pltpu.VMEM((_PER_W,), jnp.int32),
        pltpu.VMEM((_CHUNK, _HIDDEN), jnp.float32),
        pltpu.VMEM((_CHUNK, _HIDDEN), jnp.float32),
        pltpu.SemaphoreType.DMA,
        pltpu.SemaphoreType.DMA,
        pltpu.SemaphoreType.DMA,
        pltpu.SemaphoreType.DMA,
    ],
)
def _gather_rows(idx_hbm, table_hbm, out_hbm, idx_v, rows_a, rows_b,
                 gsem_a, gsem_b, osem_a, osem_b):
    wid = lax.axis_index("s") * _NC + lax.axis_index("c")
    base = wid * _PER_W
    pltpu.sync_copy(idx_hbm.at[pl.ds(base, _PER_W)], idx_v)

    bufs = (rows_a, rows_b)
    gsems = (gsem_a, gsem_b)
    osems = (osem_a, osem_b)

    g0 = pltpu.async_copy(
        table_hbm.at[idx_v.at[pl.ds(0, _CHUNK)]], bufs[0], gsems[0]
    )
    gathers = [g0, None]
    outs = [None, None]
    for i in range(_NCH):
        b = i % 2
        nb = (i + 1) % 2
        if i + 1 < _NCH:
            if outs[nb] is not None:
                outs[nb].wait()
                outs[nb] = None
            gathers[nb] = pltpu.async_copy(
                table_hbm.at[idx_v.at[pl.ds((i + 1) * _CHUNK, _CHUNK)]],
                bufs[nb],
                gsems[nb],
            )
        gathers[b].wait()
        outs[b] = pltpu.async_copy(
            bufs[b], out_hbm.at[pl.ds(base + i * _CHUNK, _CHUNK)], osems[b]
        )
    for o in outs:
        if o is not None:
            o.wait()


def kernel(input_ids, weight):
    flat_ids = input_ids.reshape(_N).astype(jnp.int32)
    out = _gather_rows(flat_ids, weight)
    return out.reshape(_BATCH, _SEQ, _HIDDEN)


# speedup vs baseline: 1.6529x; 1.0063x over previous
import functools

import jax
import jax.numpy as jnp
from jax import lax
from jax.experimental import pallas as pl
from jax.experimental.pallas import tpu as pltpu
from jax.experimental.pallas import tpu_sc as plsc

_BATCH = 4
_SEQ = 4096
_HIDDEN = 1024
_N = _BATCH * _SEQ
_NC = 2
_NS = 16
_NW = _NC * _NS
_PER_W = _N // _NW
_CHUNK = 32
_NCH = _PER_W // _CHUNK
_NBUF = 3

_mesh = plsc.VectorSubcoreMesh(
    core_axis_name="c", subcore_axis_name="s", num_cores=_NC, num_subcores=_NS
)


@functools.partial(
    pl.kernel,
    out_type=jax.ShapeDtypeStruct((_N, _HIDDEN), jnp.float32),
    mesh=_mesh,
    scratch_types=[
        pltpu.VMEM((_PER_W,), jnp.int32),
        *([pltpu.VMEM((_CHUNK, _HIDDEN), jnp.float32)] * _NBUF),
        *([pltpu.SemaphoreType.DMA] * (2 * _NBUF)),
    ],
)
def _gather_rows(idx_hbm, table_hbm, out_hbm, idx_v, *bufs_and_sems):
    bufs = bufs_and_sems[:_NBUF]
    gsems = bufs_and_sems[_NBUF:2 * _NBUF]
    osems = bufs_and_sems[2 * _NBUF:]
    wid = lax.axis_index("s") * _NC + lax.axis_index("c")
    base = wid * _PER_W
    pltpu.sync_copy(idx_hbm.at[pl.ds(base, _PER_W)], idx_v)

    def start_gather(j):
        return pltpu.async_copy(
            table_hbm.at[idx_v.at[pl.ds(j * _CHUNK, _CHUNK)]],
            bufs[j % _NBUF],
            gsems[j % _NBUF],
        )

    gathers = [None] * _NBUF
    outs = [None] * _NBUF
    for j in range(min(_NBUF, _NCH)):
        gathers[j] = start_gather(j)
    for i in range(_NCH):
        b = i % _NBUF
        gathers[b].wait()
        outs[b] = pltpu.async_copy(
            bufs[b], out_hbm.at[pl.ds(base + i * _CHUNK, _CHUNK)], osems[b]
        )
        j = i + _NBUF
        if j < _NCH:
            outs[b].wait()
            outs[b] = None
            gathers[b] = start_gather(j)
    for o in outs:
        if o is not None:
            o.wait()


def kernel(input_ids, weight):
    flat_ids = input_ids.reshape(_N).astype(jnp.int32)
    out = _gather_rows(flat_ids, weight)
    return out.reshape(_BATCH, _SEQ, _HIDDEN)
